# baseline (speedup 1.0000x reference)
import functools

import jax
import jax.numpy as jnp
from jax import lax
from jax.experimental import pallas as pl
from jax.experimental.pallas import tpu as pltpu
from jax.experimental.pallas import tpu_sc as plsc

B, H, W, C = 8, 224, 224, 96
NC, NS = 2, 16
NW = NC * NS
ROWS = B * H
RPW = ROWS // NW
HALF = W // 2
NG = HALF // 16
FH = float(H - 1)
FW = float(W - 1)


def _compute_and_fire(lane, i_f, vol_base, h, trf_v, vol_hbm, wgt, cor, sem):
    for g in range(NG):
        p0 = h * HALF + g * 16
        pidx = lane + p0
        dx = plsc.load_gather(trf_v, [pidx * 2])
        dy = plsc.load_gather(trf_v, [pidx * 2 + 1])
        x = jnp.clip(jnp.full((16,), i_f, jnp.float32) + dx, 0.0, FH)
        y = jnp.clip(pidx.astype(jnp.float32) + dy, 0.0, FW)
        x0i = x.astype(jnp.int32)
        y0i = y.astype(jnp.int32)
        x1f = jnp.minimum(x0i.astype(jnp.float32) + 1.0, FH)
        y1f = jnp.minimum(y0i.astype(jnp.float32) + 1.0, FW)
        wx1 = jnp.clip(x1f - x, 0.0, 1.0)
        wy1 = jnp.clip(y1f - y, 0.0, 1.0)
        wx0 = 1.0 - wx1
        wy0 = 1.0 - wy1
        x1i = x1f.astype(jnp.int32)
        y1i = y1f.astype(jnp.int32)
        r0 = x0i * W + vol_base
        r1 = x1i * W + vol_base
        sl = pl.ds(g * 16, 16)
        wgt[0][sl] = wx1 * wy1
        wgt[1][sl] = wx1 * wy0
        wgt[2][sl] = wx0 * wy1
        wgt[3][sl] = wx0 * wy0
        pltpu.async_copy(vol_hbm.at[r0 + y0i], cor[0].at[sl], sem)
        pltpu.async_copy(vol_hbm.at[r0 + y1i], cor[1].at[sl], sem)
        pltpu.async_copy(vol_hbm.at[r1 + y0i], cor[2].at[sl], sem)
        pltpu.async_copy(vol_hbm.at[r1 + y1i], cor[3].at[sl], sem)


def _wait(vol_hbm, cor, sem):
    zidx = jnp.zeros((16,), jnp.int32)
    for g in range(NG):
        sl = pl.ds(g * 16, 16)
        for k in range(4):
            pltpu.make_async_copy(vol_hbm.at[zidx],
                                  cor[k].at[sl], sem).wait()


def _blend(lane, cor, wgt, out_v):
    def grp_body(g, carry):
        pb = g * 16
        for l in range(16):
            p = pb + l
            pv = jnp.full((16,), p, jnp.int32)
            b0 = plsc.load_gather(wgt[0], [pv])
            b1 = plsc.load_gather(wgt[1], [pv])
            b2 = plsc.load_gather(wgt[2], [pv])
            b3 = plsc.load_gather(wgt[3], [pv])
            for v in range(C // 16):
                s = pl.ds(v * 16, 16)
                acc = cor[0][p, s] * b0 + cor[1][p, s] * b1
                acc = acc + cor[2][p, s] * b2
                acc = acc + cor[3][p, s] * b3
                out_v[p, s] = acc
        return carry

    lax.fori_loop(0, NG, grp_body, 0)


def _body(vol_hbm, trf_hbm, out_hbm, trf_v,
          wA0, wA1, wA2, wA3, wB0, wB1, wB2, wB3,
          cA0, cA1, cA2, cA3, cB0, cB1, cB2, cB3,
          outA, outB, semA, semB, semOA, semOB):
    wid = lax.axis_index("s") * NC + lax.axis_index("c")
    lane = lax.iota(jnp.int32, 16)
    wA = (wA0, wA1, wA2, wA3)
    wB = (wB0, wB1, wB2, wB3)
    cA = (cA0, cA1, cA2, cA3)
    cB = (cB0, cB1, cB2, cB3)

    def task_args(task):
        b = task // H
        i = task - b * H
        return lax.convert_element_type(i, jnp.float32), b * (H * W)

    task0 = wid * RPW
    pltpu.sync_copy(trf_hbm.at[task0], trf_v)
    i_f0, vb0 = task_args(task0)
    _compute_and_fire(lane, i_f0, vb0, 0, trf_v, vol_hbm, wA, cA, semA)

    def row_body(t, carry):
        task = wid * RPW + t
        i_f, vb = task_args(task)
        _compute_and_fire(lane, i_f, vb, 1, trf_v, vol_hbm, wB, cB, semB)
        _wait(vol_hbm, cA, semA)

        @pl.when(t > 0)
        def _():
            pltpu.make_async_copy(
                outA, out_hbm.at[pl.ds(task * W, HALF)], semOA).wait()

        _blend(lane, cA, wA, outA)
        pltpu.async_copy(outA, out_hbm.at[pl.ds(task * W, HALF)], semOA)
        tnext = jnp.minimum(task + 1, ROWS - 1)
        pltpu.sync_copy(trf_hbm.at[tnext], trf_v)
        i_fn, vbn = task_args(tnext)
        _compute_and_fire(lane, i_fn, vbn, 0, trf_v, vol_hbm, wA, cA, semA)
        _wait(vol_hbm, cB, semB)

        @pl.when(t > 0)
        def _():
            pltpu.make_async_copy(
                outB, out_hbm.at[pl.ds(task * W + HALF, HALF)], semOB).wait()

        _blend(lane, cB, wB, outB)
        pltpu.async_copy(outB, out_hbm.at[pl.ds(task * W + HALF, HALF)],
                         semOB)
        return carry

    lax.fori_loop(0, RPW, row_body, 0)
    _wait(vol_hbm, cA, semA)
    last = wid * RPW + RPW - 1
    pltpu.make_async_copy(outA, out_hbm.at[pl.ds(last * W, HALF)],
                          semOA).wait()
    pltpu.make_async_copy(outB, out_hbm.at[pl.ds(last * W + HALF, HALF)],
                          semOB).wait()


@jax.jit
def kernel(vol, trf):
    vol2 = vol.reshape(B * H * W, C)
    trf2 = trf.reshape(ROWS, W * 2)
    mesh = plsc.VectorSubcoreMesh(core_axis_name="c", subcore_axis_name="s",
                                  num_cores=NC, num_subcores=NS)
    wgt_t = [pltpu.VMEM((HALF,), jnp.float32) for _ in range(8)]
    cor_t = [pltpu.VMEM((HALF, C), jnp.float32) for _ in range(8)]
    run = functools.partial(
        pl.kernel,
        out_type=jax.ShapeDtypeStruct((B * H * W, C), jnp.float32),
        mesh=mesh,
        compiler_params=pltpu.CompilerParams(needs_layout_passes=False,
                                             use_tc_tiling_on_sc=False),
        scratch_types=(
            [pltpu.VMEM((W * 2,), jnp.float32)] + wgt_t + cor_t
            + [pltpu.VMEM((HALF, C), jnp.float32),
               pltpu.VMEM((HALF, C), jnp.float32),
               pltpu.SemaphoreType.DMA,
               pltpu.SemaphoreType.DMA,
               pltpu.SemaphoreType.DMA,
               pltpu.SemaphoreType.DMA]
        ),
    )(_body)
    return run(vol2, trf2).reshape(B, H, W, C)


# speedup vs baseline: 1.0293x; 1.0293x over previous
import functools

import jax
import jax.numpy as jnp
from jax import lax
from jax.experimental import pallas as pl
from jax.experimental.pallas import tpu as pltpu
from jax.experimental.pallas import tpu_sc as plsc

B, H, W, C = 8, 224, 224, 96
NC, NS = 2, 16
NW = NC * NS
ROWS = B * H
RPW = ROWS // NW
HALF = W // 2
NG = HALF // 16
FH = float(H - 1)
FW = float(W - 1)


def _compute_and_fire(lane, i_f, vol_base, h, trf_v, trow, vol_hbm, wgt, cor,
                      sem):
    rvec = jnp.full((16,), trow, jnp.int32)
    for g in range(NG):
        p0 = h * HALF + g * 16
        pidx = lane + p0
        dx = plsc.load_gather(trf_v, [rvec, pidx * 2])
        dy = plsc.load_gather(trf_v, [rvec, pidx * 2 + 1])
        x = jnp.clip(jnp.full((16,), i_f, jnp.float32) + dx, 0.0, FH)
        y = jnp.clip(pidx.astype(jnp.float32) + dy, 0.0, FW)
        x0i = x.astype(jnp.int32)
        y0i = y.astype(jnp.int32)
        x1f = jnp.minimum(x0i.astype(jnp.float32) + 1.0, FH)
        y1f = jnp.minimum(y0i.astype(jnp.float32) + 1.0, FW)
        wx1 = jnp.clip(x1f - x, 0.0, 1.0)
        wy1 = jnp.clip(y1f - y, 0.0, 1.0)
        wx0 = 1.0 - wx1
        wy0 = 1.0 - wy1
        x1i = x1f.astype(jnp.int32)
        y1i = y1f.astype(jnp.int32)
        r0 = x0i * W + vol_base
        r1 = x1i * W + vol_base
        sl = pl.ds(g * 16, 16)
        wgt[0][sl] = wx1 * wy1
        wgt[1][sl] = wx1 * wy0
        wgt[2][sl] = wx0 * wy1
        wgt[3][sl] = wx0 * wy0
        pltpu.async_copy(vol_hbm.at[r0 + y0i], cor[0].at[sl], sem)
        pltpu.async_copy(vol_hbm.at[r0 + y1i], cor[1].at[sl], sem)
        pltpu.async_copy(vol_hbm.at[r1 + y0i], cor[2].at[sl], sem)
        pltpu.async_copy(vol_hbm.at[r1 + y1i], cor[3].at[sl], sem)


def _wait(vol_hbm, cor, sem):
    zidx = jnp.zeros((16,), jnp.int32)
    for g in range(NG):
        sl = pl.ds(g * 16, 16)
        for k in range(4):
            pltpu.make_async_copy(vol_hbm.at[zidx],
                                  cor[k].at[sl], sem).wait()


def _blend(lane, cor, wgt, out_v):
    def grp_body(g, carry):
        pb = g * 16
        for l in range(16):
            p = pb + l
            pv = jnp.full((16,), p, jnp.int32)
            b0 = plsc.load_gather(wgt[0], [pv])
            b1 = plsc.load_gather(wgt[1], [pv])
            b2 = plsc.load_gather(wgt[2], [pv])
            b3 = plsc.load_gather(wgt[3], [pv])
            for v in range(C // 16):
                s = pl.ds(v * 16, 16)
                acc = cor[0][p, s] * b0 + cor[1][p, s] * b1
                acc = acc + cor[2][p, s] * b2
                acc = acc + cor[3][p, s] * b3
                out_v[p, s] = acc
        return carry

    lax.fori_loop(0, NG, grp_body, 0)


def _body(vol_hbm, trf_hbm, out_hbm, trf_v,
          wA0, wA1, wA2, wA3, wB0, wB1, wB2, wB3,
          cA0, cA1, cA2, cA3, cB0, cB1, cB2, cB3,
          outA, outB, semA, semB, semOA, semOB):
    wid = lax.axis_index("s") * NC + lax.axis_index("c")
    lane = lax.iota(jnp.int32, 16)
    wA = (wA0, wA1, wA2, wA3)
    wB = (wB0, wB1, wB2, wB3)
    cA = (cA0, cA1, cA2, cA3)
    cB = (cB0, cB1, cB2, cB3)

    def task_args(task):
        b = task // H
        i = task - b * H
        return lax.convert_element_type(i, jnp.float32), b * (H * W)

    task0 = wid * RPW
    pltpu.sync_copy(trf_hbm.at[pl.ds(task0, 28)], trf_v)
    i_f0, vb0 = task_args(task0)
    _compute_and_fire(lane, i_f0, vb0, 0, trf_v, 0, vol_hbm, wA, cA, semA)

    def row_body(t, carry):
        task = wid * RPW + t
        i_f, vb = task_args(task)
        trow = t - (t // 28) * 28
        _compute_and_fire(lane, i_f, vb, 1, trf_v, trow, vol_hbm, wB, cB,
                          semB)
        _wait(vol_hbm, cA, semA)

        @pl.when(t > 0)
        def _():
            pltpu.make_async_copy(
                outA, out_hbm.at[pl.ds(task * W, HALF)], semOA).wait()

        _blend(lane, cA, wA, outA)
        pltpu.async_copy(outA, out_hbm.at[pl.ds(task * W, HALF)], semOA)
        tn = t + 1
        trow_n = tn - (tn // 28) * 28

        @pl.when(jnp.logical_and(trow_n == 0, tn < RPW))
        def _():
            pltpu.sync_copy(trf_hbm.at[pl.ds(wid * RPW + tn, 28)], trf_v)

        tnext = jnp.minimum(task + 1, ROWS - 1)
        i_fn, vbn = task_args(tnext)
        _compute_and_fire(lane, i_fn, vbn, 0, trf_v,
                          jnp.minimum(trow_n, 27), vol_hbm, wA, cA, semA)
        _wait(vol_hbm, cB, semB)

        @pl.when(t > 0)
        def _():
            pltpu.make_async_copy(
                outB, out_hbm.at[pl.ds(task * W + HALF, HALF)], semOB).wait()

        _blend(lane, cB, wB, outB)
        pltpu.async_copy(outB, out_hbm.at[pl.ds(task * W + HALF, HALF)],
                         semOB)
        return carry

    lax.fori_loop(0, RPW, row_body, 0)
    _wait(vol_hbm, cA, semA)
    last = wid * RPW + RPW - 1
    pltpu.make_async_copy(outA, out_hbm.at[pl.ds(last * W, HALF)],
                          semOA).wait()
    pltpu.make_async_copy(outB, out_hbm.at[pl.ds(last * W + HALF, HALF)],
                          semOB).wait()


@jax.jit
def kernel(vol, trf):
    vol2 = vol.reshape(B * H * W, C)
    trf2 = trf.reshape(ROWS, W * 2)
    mesh = plsc.VectorSubcoreMesh(core_axis_name="c", subcore_axis_name="s",
                                  num_cores=NC, num_subcores=NS)
    wgt_t = [pltpu.VMEM((HALF,), jnp.float32) for _ in range(8)]
    cor_t = [pltpu.VMEM((HALF, C), jnp.float32) for _ in range(8)]
    run = functools.partial(
        pl.kernel,
        out_type=jax.ShapeDtypeStruct((B * H * W, C), jnp.float32),
        mesh=mesh,
        compiler_params=pltpu.CompilerParams(needs_layout_passes=False,
                                             use_tc_tiling_on_sc=False),
        scratch_types=(
            [pltpu.VMEM((28, W * 2), jnp.float32)] + wgt_t + cor_t
            + [pltpu.VMEM((HALF, C), jnp.float32),
               pltpu.VMEM((HALF, C), jnp.float32),
               pltpu.SemaphoreType.DMA,
               pltpu.SemaphoreType.DMA,
               pltpu.SemaphoreType.DMA,
               pltpu.SemaphoreType.DMA]
        ),
    )(_body)
    return run(vol2, trf2).reshape(B, H, W, C)
